# Initial kernel scaffold; baseline (speedup 1.0000x reference)
import jax
import jax.numpy as jnp
from jax.experimental import pallas as pl


def kernel(x, Wg, W1, b1, W2, b2):
    raise NotImplementedError("write your pallas kernel here")


# speedup vs baseline: 2.1601x; 2.1601x over previous
import functools

import jax
import jax.numpy as jnp
from jax.experimental import pallas as pl
from jax.experimental.pallas import tpu as pltpu

D = 1024
E = 8
DFF = 4096
TOPK = 2

TM = 256
FC = 1024
NF = DFF // FC


def _gmm_body(eid_ref, x_ref, w1_ref, b1_ref, w2_ref, b2_ref, o_ref):
    f = pl.program_id(1)
    h = jnp.dot(x_ref[...], w1_ref[0], preferred_element_type=jnp.float32)
    h = jax.nn.gelu(h + b1_ref[0])
    contrib = jnp.dot(h, w2_ref[0], preferred_element_type=jnp.float32)

    @pl.when(f == 0)
    def _():
        o_ref[...] = contrib + b2_ref[0]

    @pl.when(f != 0)
    def _():
        o_ref[...] = o_ref[...] + contrib


def _grouped_mlp(xs, W1, b1, W2, b2, eid_tile, nt):
    grid_spec = pltpu.PrefetchScalarGridSpec(
        num_scalar_prefetch=1,
        grid=(nt, NF),
        in_specs=[
            pl.BlockSpec((TM, D), lambda t, f, eid: (t, 0)),
            pl.BlockSpec((1, D, FC), lambda t, f, eid: (eid[t], 0, f)),
            pl.BlockSpec((1, 1, FC), lambda t, f, eid: (eid[t], 0, f)),
            pl.BlockSpec((1, FC, D), lambda t, f, eid: (eid[t], f, 0)),
            pl.BlockSpec((1, 1, D), lambda t, f, eid: (eid[t], 0, 0)),
        ],
        out_specs=pl.BlockSpec((TM, D), lambda t, f, eid: (t, 0)),
    )
    return pl.pallas_call(
        _gmm_body,
        grid_spec=grid_spec,
        out_shape=jax.ShapeDtypeStruct((nt * TM, D), jnp.float32),
        compiler_params=pltpu.CompilerParams(
            dimension_semantics=("arbitrary", "arbitrary"),
        ),
    )(eid_tile, xs, W1, b1.reshape(E, 1, DFF), W2, b2.reshape(E, 1, D))


def kernel(x, Wg, W1, b1, W2, b2):
    flat = x.reshape(-1, D)
    t_tokens = flat.shape[0]

    scores = flat @ Wg
    ew, ei = jax.lax.top_k(scores, TOPK)
    ew = jax.nn.softmax(ew, axis=-1)

    flat_ei = ei.reshape(-1)
    oh = (flat_ei[:, None] == jnp.arange(E)[None, :]).astype(jnp.int32)
    cum = jnp.cumsum(oh, axis=0)
    rank = jnp.sum((cum - oh) * oh, axis=1)
    counts = cum[-1]

    padded = ((counts + TM - 1) // TM) * TM
    offs = jnp.concatenate([jnp.zeros((1,), jnp.int32),
                            jnp.cumsum(padded)[:-1].astype(jnp.int32)])
    dest = offs[flat_ei] + rank

    ntot = t_tokens * TOPK + (E - 1) * TM
    nt = ntot // TM
    tiles_per_e = (padded // TM).astype(jnp.int32)
    eid_tile = jnp.repeat(jnp.arange(E, dtype=jnp.int32), tiles_per_e,
                          total_repeat_length=nt)

    xs = jnp.zeros((ntot, D), jnp.float32).at[dest].set(
        jnp.repeat(flat, TOPK, axis=0))

    ys = _grouped_mlp(xs, W1, b1, W2, b2, eid_tile, nt)

    gathered = ys[dest.reshape(t_tokens, TOPK)]
    y = jnp.sum(gathered * ew[:, :, None], axis=1)
    return y
